# Initial kernel scaffold; baseline (speedup 1.0000x reference)
import jax
import jax.numpy as jnp
from jax.experimental import pallas as pl


def kernel(x, expert_embs):
    raise NotImplementedError("write your pallas kernel here")


# speedup vs baseline: 5.6630x; 5.6630x over previous
import functools

import jax
import jax.numpy as jnp
from jax.experimental import pallas as pl

B, S, D = 4, 4096, 4096
NUM_EXPERTS = 64
K = 8
ROWS = B * S
BLK = 512


def _router_block(x_ref, w_ref, probs_ref, masks_t_ref):
    s = jnp.dot(x_ref[...], w_ref[...], preferred_element_type=jnp.float32)
    m = jnp.max(s, axis=-1, keepdims=True)
    e = jnp.exp(s - m)
    p = e / jnp.sum(e, axis=-1, keepdims=True)
    work = s
    for _ in range(K):
        t = jnp.max(work, axis=-1, keepdims=True)
        work = jnp.where(work == t, -jnp.inf, work)
    keep = s >= t
    probs_ref[...] = p
    masks_t_ref[...] = jnp.where(keep, p, 0.0).T


@functools.partial(jax.jit, static_argnums=())
def kernel(x, expert_embs):
    xf = x.reshape(ROWS, D)
    grid = (ROWS // BLK,)
    probs, masks_t = pl.pallas_call(
        _router_block,
        grid=grid,
        in_specs=[
            pl.BlockSpec((BLK, D), lambda i: (i, 0)),
            pl.BlockSpec((D, NUM_EXPERTS), lambda i: (0, 0)),
        ],
        out_specs=[
            pl.BlockSpec((BLK, NUM_EXPERTS), lambda i: (i, 0)),
            pl.BlockSpec((NUM_EXPERTS, BLK), lambda i: (0, i)),
        ],
        out_shape=[
            jax.ShapeDtypeStruct((ROWS, NUM_EXPERTS), jnp.float32),
            jax.ShapeDtypeStruct((NUM_EXPERTS, ROWS), jnp.float32),
        ],
    )(xf, expert_embs)
    experts_masks = masks_t.reshape(NUM_EXPERTS, B, S, 1)
    aux_loss = jnp.zeros((), jnp.float32)
    return (experts_masks, aux_loss, probs)
